# Initial kernel scaffold; baseline (speedup 1.0000x reference)
import jax
import jax.numpy as jnp
from jax.experimental import pallas as pl


def kernel(node_features, edge_index, node_pos, edge_attr, params):
    raise NotImplementedError("write your pallas kernel here")


# speedup vs baseline: 1.5399x; 1.5399x over previous
import functools

import jax
import jax.numpy as jnp
from jax import lax
from jax.experimental import pallas as pl
from jax.experimental.pallas import tpu as pltpu
from jax.experimental.pallas import tpu_sc as plsc

F32 = jnp.float32
PREC = jax.lax.Precision.HIGHEST
NC, NS = 2, 16
NW = NC * NS


def _silu(x):
    return x * jax.nn.sigmoid(x)


def _linear_tc(h, w, b, bn):
    n, k = h.shape
    m = w.shape[1]
    nb = n // bn

    def body(h_ref, w_ref, b_ref, o_ref):
        o_ref[...] = (
            jnp.dot(h_ref[...], w_ref[...], preferred_element_type=F32, precision=PREC)
            + b_ref[...]
        )

    return pl.pallas_call(
        body,
        grid=(nb,),
        in_specs=[
            pl.BlockSpec((bn, k), lambda i: (i, 0)),
            pl.BlockSpec((k, m), lambda i: (0, 0)),
            pl.BlockSpec((1, m), lambda i: (0, 0)),
        ],
        out_specs=pl.BlockSpec((bn, m), lambda i: (i, 0)),
        out_shape=jax.ShapeDtypeStruct((n, m), F32),
    )(h, w, b.reshape(1, -1))


def _pq_tc(h, wab, bab, coord, bn):
    n, k = h.shape
    m = wab.shape[2]
    nb = n // bn

    def body(h_ref, w_ref, b_ref, c_ref, o_ref):
        o_ref[:, :m] = (
            jnp.dot(h_ref[...], w_ref[0], preferred_element_type=F32, precision=PREC)
            + b_ref[0]
        )
        o_ref[:, m:] = c_ref[...]

    return pl.pallas_call(
        body,
        grid=(2, nb),
        in_specs=[
            pl.BlockSpec((bn, k), lambda j, i: (i, 0)),
            pl.BlockSpec((1, k, m), lambda j, i: (j, 0, 0)),
            pl.BlockSpec((1, 1, m), lambda j, i: (j, 0, 0)),
            pl.BlockSpec((bn, 128), lambda j, i: (i, 0)),
        ],
        out_specs=pl.BlockSpec((bn, m + 128), lambda j, i: (j * nb + i, 0)),
        out_shape=jax.ShapeDtypeStruct((2 * n, m + 128), F32),
    )(h, wab, bab, coord)


def _edge_tc(g2, ea, wr, we, w2, b2, watt, batt, wc1, bc1, wc2, e, hdim, be):
    nbe = e // be
    w = hdim + 128

    def body(gr_ref, gc_ref, ea_ref, wr_ref, we_ref, w2_ref,
             b2_ref, watt_ref, batt_ref, wc1_ref, bc1_ref, wc2_ref,
             m_ref, t_ref):
        gr = gr_ref[:, :hdim]
        gc = gc_ref[:, :hdim]
        xr = gr_ref[:, hdim:]
        xc = gc_ref[:, hdim:]
        diff = xr - xc
        radial = jnp.sum(diff * diff, axis=1, keepdims=True)
        dn = diff / (jnp.sqrt(radial) + 1e-8)
        t = (gr + gc
             + radial * wr_ref[...]
             + jnp.dot(ea_ref[...], we_ref[...], preferred_element_type=F32, precision=PREC))
        m1 = _silu(t)
        m2 = jnp.dot(m1, w2_ref[...], preferred_element_type=F32, precision=PREC) + b2_ref[...]
        m = _silu(m2)
        attl = jnp.sum(m * watt_ref[...], axis=1, keepdims=True) + batt_ref[...]
        m = m * jax.nn.sigmoid(attl)
        cm = jnp.dot(m, wc1_ref[...], preferred_element_type=F32, precision=PREC) + bc1_ref[...]
        cm = _silu(cm)
        s = jnp.sum(cm * wc2_ref[...], axis=1, keepdims=True)
        e3 = (lax.broadcasted_iota(jnp.int32, (1, 128), 1) == 3).astype(F32)
        m_ref[...] = m
        t_ref[...] = dn * s + e3

    cw = lambda shp: pl.BlockSpec(shp, lambda i: tuple(0 for _ in shp))
    return pl.pallas_call(
        body,
        grid=(nbe,),
        in_specs=[
            pl.BlockSpec((be, w), lambda i: (i, 0)),
            pl.BlockSpec((be, w), lambda i: (nbe + i, 0)),
            pl.BlockSpec((be, 16), lambda i: (i, 0)),
            cw((1, hdim)), cw((16, hdim)), cw((hdim, hdim)), cw((1, hdim)),
            cw((1, hdim)), cw((1, 1)), cw((hdim, hdim)), cw((1, hdim)),
            cw((1, hdim)),
        ],
        out_specs=[
            pl.BlockSpec((be, hdim), lambda i: (i, 0)),
            pl.BlockSpec((be, 128), lambda i: (i, 0)),
        ],
        out_shape=[
            jax.ShapeDtypeStruct((e, hdim), F32),
            jax.ShapeDtypeStruct((e, 128), F32),
        ],
    )(g2, g2, ea, wr, we, w2, b2, watt, batt, wc1, bc1, wc2)


def _node_tc(h, agg, tsum, coord, wn1h, wn1a, bn1, wn2, bn2, bn):
    n, hdim = h.shape
    p = agg.shape[0] // n
    nb = n // bn

    def body(*refs):
        h_ref = refs[0]
        aggs = refs[1:1 + p]
        t0, t1 = refs[1 + p], refs[2 + p]
        coord_ref = refs[3 + p]
        wn1h_ref, wn1a_ref, bn1_ref, wn2_ref, bn2_ref = refs[4 + p:9 + p]
        h_out, c_out = refs[9 + p], refs[10 + p]

        a = aggs[0][...]
        if p == 2:
            a = a + aggs[1][...]
        ts = t0[...] + t1[...]
        lane = lax.broadcasted_iota(jnp.int32, (1, 128), 1)
        mask3 = (lane < 3).astype(F32)
        e3 = (lane == 3).astype(F32)
        cnt = jnp.sum(ts * e3, axis=1, keepdims=True)
        inv = 1.0 / jnp.maximum(cnt, 1.0)
        c_out[...] = coord_ref[...] + ts * mask3 * inv
        tmp = (jnp.dot(h_ref[...], wn1h_ref[...], preferred_element_type=F32, precision=PREC)
               + jnp.dot(a, wn1a_ref[...], preferred_element_type=F32, precision=PREC)
               + bn1_ref[...])
        tmp = _silu(tmp)
        h_out[...] = (h_ref[...]
                      + jnp.dot(tmp, wn2_ref[...], preferred_element_type=F32, precision=PREC)
                      + bn2_ref[...])

    cw = lambda shp: pl.BlockSpec(shp, lambda i: tuple(0 for _ in shp))
    in_specs = [pl.BlockSpec((bn, hdim), lambda i: (i, 0))]
    for q in range(p):
        in_specs.append(pl.BlockSpec((bn, hdim), lambda i, q=q: (q * nb + i, 0)))
    in_specs += [
        pl.BlockSpec((bn, 128), lambda i: (i, 0)),
        pl.BlockSpec((bn, 128), lambda i: (nb + i, 0)),
        pl.BlockSpec((bn, 128), lambda i: (i, 0)),
        cw((hdim, hdim)), cw((hdim, hdim)), cw((1, hdim)),
        cw((hdim, hdim)), cw((1, hdim)),
    ]
    return pl.pallas_call(
        body,
        grid=(nb,),
        in_specs=in_specs,
        out_specs=[
            pl.BlockSpec((bn, hdim), lambda i: (i, 0)),
            pl.BlockSpec((bn, 128), lambda i: (i, 0)),
        ],
        out_shape=[
            jax.ShapeDtypeStruct((n, hdim), F32),
            jax.ShapeDtypeStruct((n, 128), F32),
        ],
    )(h, *([agg] * p), tsum, tsum, coord, wn1h, wn1a, bn1.reshape(1, -1),
      wn2, bn2.reshape(1, -1))


def _final_tc(h, w, b, bn):
    n, k = h.shape
    m = w.shape[1]
    nb = n // bn

    def body(h_ref, w_ref, b_ref, o_ref, acc_ref):
        i = pl.program_id(0)

        @pl.when(i == 0)
        def _():
            acc_ref[...] = jnp.zeros_like(acc_ref)

        acc_ref[...] += jnp.sum(h_ref[...], axis=0, keepdims=True)

        @pl.when(i == nb - 1)
        def _():
            o_ref[...] = (
                jnp.dot(acc_ref[...] / n, w_ref[...], preferred_element_type=F32, precision=PREC)
                + b_ref[...]
            )

    return pl.pallas_call(
        body,
        grid=(nb,),
        in_specs=[
            pl.BlockSpec((bn, k), lambda i: (i, 0)),
            pl.BlockSpec((k, m), lambda i: (0, 0)),
            pl.BlockSpec((1, m), lambda i: (0, 0)),
        ],
        out_specs=pl.BlockSpec((1, m), lambda i: (0, 0)),
        out_shape=jax.ShapeDtypeStruct((1, m), F32),
        scratch_shapes=[pltpu.VMEM((1, k), F32)],
    )(h, w, b.reshape(1, -1))


def _sc_gather(table, idx, ch):
    v, d = table.shape
    b = idx.shape[0]
    assert b % NW == 0
    bpw = b // NW
    assert bpw % ch == 0 and ch % 8 == 0
    nit = bpw // ch
    mesh = plsc.VectorSubcoreMesh(core_axis_name="c", subcore_axis_name="s")

    @functools.partial(
        pl.kernel,
        mesh=mesh,
        out_type=jax.ShapeDtypeStruct((b, d), F32),
        scratch_types=[
            pltpu.VMEM((ch,), jnp.int32),
            pltpu.VMEM((ch, d), F32),
            pltpu.SemaphoreType.DMA,
        ],
    )
    def k(table_hbm, idx_hbm, out_hbm, idx_v, rows_v, sem):
        wid = lax.axis_index("s") * NC + lax.axis_index("c")
        base0 = wid * bpw

        def step(i, carry):
            base = pl.multiple_of(base0 + i * ch, 8)
            pltpu.sync_copy(idx_hbm.at[pl.ds(base, ch)], idx_v)
            pltpu.async_copy(table_hbm.at[idx_v], rows_v, sem).wait()
            pltpu.sync_copy(rows_v, out_hbm.at[pl.ds(base, ch)])
            return carry

        lax.fori_loop(0, nit, step, 0)

    return k(table, idx)


def _sc_scatter_cols(m, idx, zeros, n, c, bet):
    e, hdim = m.shape
    nck = hdim // c
    ncps = nck // NC
    ept = e // NS
    assert ept % bet == 0 and bet % 8 == 0
    nit = ept // bet
    rpt = (-(-n // NS) + 7) // 8 * 8
    tail = n - (NS - 1) * rpt
    assert tail > 0 and tail % 8 == 0
    mesh = plsc.VectorSubcoreMesh(core_axis_name="c", subcore_axis_name="s")

    @functools.partial(
        pl.kernel,
        mesh=mesh,
        out_type=jax.ShapeDtypeStruct((n, hdim), F32),
        scratch_types=[
            pltpu.VMEM((bet, c), F32),
            pltpu.VMEM((bet,), jnp.int32),
            pltpu.VMEM_SHARED((n, c), F32),
        ],
    )
    def k(m_hbm, idx_hbm, z_hbm, out_hbm, mbuf, ibuf, acc):
        core = lax.axis_index("c")
        s = lax.axis_index("s")
        r0 = pl.multiple_of(s * rpt, 8)
        for kk in range(ncps):
            chunk = core + NC * kk
            c0 = chunk * c

            @pl.when(s < NS - 1)
            def _():
                pltpu.sync_copy(z_hbm.at[pl.ds(0, rpt), pl.ds(0, c)],
                                acc.at[pl.ds(r0, rpt)])

            @pl.when(s == NS - 1)
            def _():
                pltpu.sync_copy(z_hbm.at[pl.ds(0, tail), pl.ds(0, c)],
                                acc.at[pl.ds(r0, tail)])

            plsc.subcore_barrier()

            def step(i, carry):
                e0 = pl.multiple_of(s * ept + i * bet, 8)
                pltpu.sync_copy(idx_hbm.at[pl.ds(e0, bet)], ibuf)
                pltpu.sync_copy(m_hbm.at[pl.ds(e0, bet), pl.ds(c0, c)], mbuf)
                pltpu.sync_copy(mbuf, acc.at[ibuf], add=True)
                return carry

            lax.fori_loop(0, nit, step, 0)
            plsc.subcore_barrier()

            @pl.when(s < NS - 1)
            def _():
                pltpu.sync_copy(acc.at[pl.ds(r0, rpt)],
                                out_hbm.at[pl.ds(r0, rpt), pl.ds(c0, c)])

            @pl.when(s == NS - 1)
            def _():
                pltpu.sync_copy(acc.at[pl.ds(r0, tail)],
                                out_hbm.at[pl.ds(r0, tail), pl.ds(c0, c)])

            plsc.subcore_barrier()

    return k(m, idx, zeros)


def _sc_scatter_halves(m, idx, zeros, n, bet):
    e, c = m.shape
    eps = e // NC
    ept = eps // NS
    assert ept % bet == 0 and bet % 8 == 0
    nit = ept // bet
    rpt = (-(-n // NS) + 7) // 8 * 8
    tail = n - (NS - 1) * rpt
    assert tail > 0 and tail % 8 == 0
    mesh = plsc.VectorSubcoreMesh(core_axis_name="c", subcore_axis_name="s")

    @functools.partial(
        pl.kernel,
        mesh=mesh,
        out_type=jax.ShapeDtypeStruct((2 * n, c), F32),
        scratch_types=[
            pltpu.VMEM((bet, c), F32),
            pltpu.VMEM((bet,), jnp.int32),
            pltpu.VMEM_SHARED((n, c), F32),
        ],
    )
    def k(m_hbm, idx_hbm, z_hbm, out_hbm, mbuf, ibuf, acc):
        core = lax.axis_index("c")
        s = lax.axis_index("s")
        r0 = pl.multiple_of(s * rpt, 8)
        o0 = pl.multiple_of(core * n + s * rpt, 8)

        @pl.when(s < NS - 1)
        def _():
            pltpu.sync_copy(z_hbm.at[pl.ds(0, rpt), pl.ds(0, c)],
                            acc.at[pl.ds(r0, rpt)])

        @pl.when(s == NS - 1)
        def _():
            pltpu.sync_copy(z_hbm.at[pl.ds(0, tail), pl.ds(0, c)],
                            acc.at[pl.ds(r0, tail)])

        plsc.subcore_barrier()

        def step(i, carry):
            e0 = pl.multiple_of(core * eps + s * ept + i * bet, 8)
            pltpu.sync_copy(idx_hbm.at[pl.ds(e0, bet)], ibuf)
            pltpu.sync_copy(m_hbm.at[pl.ds(e0, bet), pl.ds(0, c)], mbuf)
            pltpu.sync_copy(mbuf, acc.at[ibuf], add=True)
            return carry

        lax.fori_loop(0, nit, step, 0)
        plsc.subcore_barrier()

        @pl.when(s < NS - 1)
        def _():
            pltpu.sync_copy(acc.at[pl.ds(r0, rpt)],
                            out_hbm.at[pl.ds(o0, rpt), pl.ds(0, c)])

        @pl.when(s == NS - 1)
        def _():
            pltpu.sync_copy(acc.at[pl.ds(r0, tail)],
                            out_hbm.at[pl.ds(o0, tail), pl.ds(0, c)])

    return k(m, idx, zeros)


def _egcl(h, coord, lp, idx_pq, row, ea16, zeros, n, e, hdim,
          bn, be, ch_pq, bet_m, cols_mode):
    w1 = lp["edge1"]["w"]
    wa, wb = w1[:hdim], w1[hdim:2 * hdim]
    wr = w1[2 * hdim:2 * hdim + 1]
    we = jnp.pad(w1[2 * hdim + 1:], ((0, 5), (0, 0)))
    b1 = lp["edge1"]["b"]

    pq = _pq_tc(h, jnp.stack([wa, wb]),
                jnp.stack([b1.reshape(1, -1), jnp.zeros((1, hdim), F32)]),
                coord, bn)
    g2 = _sc_gather(pq, idx_pq, ch_pq)

    m, trans = _edge_tc(
        g2, ea16, wr, we, lp["edge2"]["w"], lp["edge2"]["b"].reshape(1, -1),
        lp["att"]["w"].reshape(1, -1), lp["att"]["b"].reshape(1, 1),
        lp["coord1"]["w"], lp["coord1"]["b"].reshape(1, -1),
        lp["coord2_w"].reshape(1, -1), e, hdim, be)

    if cols_mode:
        agg = _sc_scatter_cols(m, row, zeros, n, 128, bet_m)
    else:
        agg = _sc_scatter_halves(m, row, zeros, n, bet_m)
    tsum = _sc_scatter_halves(trans, row, zeros, n, 200)

    wn1 = lp["node1"]["w"]
    h, coord = _node_tc(h, agg, tsum, coord, wn1[:hdim], wn1[hdim:],
                        lp["node1"]["b"], lp["node2"]["w"], lp["node2"]["b"],
                        bn)
    return h, coord


def kernel(node_features, edge_index, node_pos, edge_attr, params):
    n = node_features.shape[0]
    e = edge_index.shape[1]
    row = edge_index[0]
    col = edge_index[1]
    idx_pq = jnp.concatenate([row, col + n])
    ea16 = jnp.pad(edge_attr, ((0, 0), (0, 16 - edge_attr.shape[1])))
    zeros = jnp.zeros((n, 128), F32)
    coord = jnp.pad(node_pos, ((0, 0), (0, 125)))
    bn = 1000

    p1 = params["egnn1"]
    h = _linear_tc(node_features, p1["emb_in"]["w"], p1["emb_in"]["b"], bn)
    for lp in p1["layers"]:
        h, coord = _egcl(h, coord, lp, idx_pq, row, ea16, zeros,
                         n, e, 512, bn, 640, 80, 200, cols_mode=True)
    h = _linear_tc(h, p1["emb_out"]["w"], p1["emb_out"]["b"], bn)

    p2 = params["egnn2"]
    h = _linear_tc(h, p2["emb_in"]["w"], p2["emb_in"]["b"], bn)
    for lp in p2["layers"]:
        h, coord = _egcl(h, coord, lp, idx_pq, row, ea16, zeros,
                         n, e, 128, bn, 2000, 200, 200, cols_mode=False)
    out = _final_tc(h, p2["emb_out"]["w"], p2["emb_out"]["b"], bn)
    return out


# speedup vs baseline: 2.5387x; 1.6486x over previous
import functools

import jax
import jax.numpy as jnp
from jax import lax
from jax.experimental import pallas as pl
from jax.experimental.pallas import tpu as pltpu
from jax.experimental.pallas import tpu_sc as plsc

F32 = jnp.float32
BF16 = jnp.bfloat16
PREC = jax.lax.Precision.HIGHEST


def _bdot(a, b):
    return jnp.dot(a.astype(BF16), b.astype(BF16),
                   preferred_element_type=F32)


def _br(x):
    return x.astype(BF16).astype(F32)
NC, NS = 2, 16
NW = NC * NS


def _silu(x):
    return x * jax.nn.sigmoid(x)


def _linear_tc(h, w, b, bn):
    n, k = h.shape
    m = w.shape[1]
    nb = n // bn

    def body(h_ref, w_ref, b_ref, o_ref):
        o_ref[...] = (
            _bdot(h_ref[...], w_ref[...])
            + b_ref[...]
        )

    return pl.pallas_call(
        body,
        grid=(nb,),
        in_specs=[
            pl.BlockSpec((bn, k), lambda i: (i, 0)),
            pl.BlockSpec((k, m), lambda i: (0, 0)),
            pl.BlockSpec((1, m), lambda i: (0, 0)),
        ],
        out_specs=pl.BlockSpec((bn, m), lambda i: (i, 0)),
        out_shape=jax.ShapeDtypeStruct((n, m), F32),
    )(h, w, b.reshape(1, -1))


def _pq_tc(h, wab, bab, coord, bn):
    n, k = h.shape
    m = wab.shape[2]
    nb = n // bn

    def body(h_ref, w_ref, b_ref, c_ref, o_ref):
        o_ref[:, :m] = (
            _bdot(h_ref[...], w_ref[0])
            + b_ref[0]
        )
        o_ref[:, m:] = c_ref[...]

    return pl.pallas_call(
        body,
        grid=(2, nb),
        in_specs=[
            pl.BlockSpec((bn, k), lambda j, i: (i, 0)),
            pl.BlockSpec((1, k, m), lambda j, i: (j, 0, 0)),
            pl.BlockSpec((1, 1, m), lambda j, i: (j, 0, 0)),
            pl.BlockSpec((bn, 128), lambda j, i: (i, 0)),
        ],
        out_specs=pl.BlockSpec((bn, m + 128), lambda j, i: (j * nb + i, 0)),
        out_shape=jax.ShapeDtypeStruct((2 * n, m + 128), F32),
    )(h, wab, bab, coord)


def _edge_tc(g2, ea, wr, we, w2, b2, watt, batt, wc1, bc1, wc2, e, hdim, be):
    nbe = e // be
    w = hdim + 128

    def body(gr_ref, gc_ref, ea_ref, wr_ref, we_ref, w2_ref,
             b2_ref, watt_ref, batt_ref, wc1_ref, bc1_ref, wc2_ref,
             m_ref, t_ref):
        gr = gr_ref[:, :hdim]
        gc = gc_ref[:, :hdim]
        xr = gr_ref[:, hdim:]
        xc = gc_ref[:, hdim:]
        diff = xr - xc
        radial = jnp.sum(diff * diff, axis=1, keepdims=True)
        dn = diff / (jnp.sqrt(radial) + 1e-8)
        t = (gr + gc
             + _br(radial) * _br(wr_ref[...])
             + _bdot(ea_ref[...], we_ref[...]))
        m1 = _silu(t)
        m2 = _bdot(m1, w2_ref[...]) + b2_ref[...]
        m = _silu(m2)
        attl = (jnp.sum(_br(m) * _br(watt_ref[...]), axis=1, keepdims=True)
                + batt_ref[...])
        m = m * jax.nn.sigmoid(attl)
        cm = _bdot(m, wc1_ref[...]) + bc1_ref[...]
        cm = _silu(cm)
        s = jnp.sum(_br(cm) * _br(wc2_ref[...]), axis=1, keepdims=True)
        e3 = (lax.broadcasted_iota(jnp.int32, (1, 128), 1) == 3).astype(F32)
        m_ref[...] = m
        t_ref[...] = dn * s + e3

    cw = lambda shp: pl.BlockSpec(shp, lambda i: tuple(0 for _ in shp))
    return pl.pallas_call(
        body,
        grid=(nbe,),
        in_specs=[
            pl.BlockSpec((be, w), lambda i: (i, 0)),
            pl.BlockSpec((be, w), lambda i: (nbe + i, 0)),
            pl.BlockSpec((be, 16), lambda i: (i, 0)),
            cw((1, hdim)), cw((16, hdim)), cw((hdim, hdim)), cw((1, hdim)),
            cw((1, hdim)), cw((1, 1)), cw((hdim, hdim)), cw((1, hdim)),
            cw((1, hdim)),
        ],
        out_specs=[
            pl.BlockSpec((be, hdim), lambda i: (i, 0)),
            pl.BlockSpec((be, 128), lambda i: (i, 0)),
        ],
        out_shape=[
            jax.ShapeDtypeStruct((e, hdim), F32),
            jax.ShapeDtypeStruct((e, 128), F32),
        ],
    )(g2, g2, ea, wr, we, w2, b2, watt, batt, wc1, bc1, wc2)


def _node_tc(h, agg, tsum, coord, wn1h, wn1a, bn1, wn2, bn2, bn):
    n, hdim = h.shape
    p = agg.shape[0] // n
    nb = n // bn

    def body(*refs):
        h_ref = refs[0]
        aggs = refs[1:1 + p]
        t0, t1 = refs[1 + p], refs[2 + p]
        coord_ref = refs[3 + p]
        wn1h_ref, wn1a_ref, bn1_ref, wn2_ref, bn2_ref = refs[4 + p:9 + p]
        h_out, c_out = refs[9 + p], refs[10 + p]

        a = aggs[0][...]
        if p == 2:
            a = a + aggs[1][...]
        ts = t0[...] + t1[...]
        lane = lax.broadcasted_iota(jnp.int32, (1, 128), 1)
        mask3 = (lane < 3).astype(F32)
        e3 = (lane == 3).astype(F32)
        cnt = jnp.sum(ts * e3, axis=1, keepdims=True)
        inv = 1.0 / jnp.maximum(cnt, 1.0)
        c_out[...] = coord_ref[...] + ts * mask3 * inv
        tmp = (_bdot(h_ref[...], wn1h_ref[...])
               + _bdot(a, wn1a_ref[...])
               + bn1_ref[...])
        tmp = _silu(tmp)
        h_out[...] = (h_ref[...]
                      + _bdot(tmp, wn2_ref[...])
                      + bn2_ref[...])

    cw = lambda shp: pl.BlockSpec(shp, lambda i: tuple(0 for _ in shp))
    in_specs = [pl.BlockSpec((bn, hdim), lambda i: (i, 0))]
    for q in range(p):
        in_specs.append(pl.BlockSpec((bn, hdim), lambda i, q=q: (q * nb + i, 0)))
    in_specs += [
        pl.BlockSpec((bn, 128), lambda i: (i, 0)),
        pl.BlockSpec((bn, 128), lambda i: (nb + i, 0)),
        pl.BlockSpec((bn, 128), lambda i: (i, 0)),
        cw((hdim, hdim)), cw((hdim, hdim)), cw((1, hdim)),
        cw((hdim, hdim)), cw((1, hdim)),
    ]
    return pl.pallas_call(
        body,
        grid=(nb,),
        in_specs=in_specs,
        out_specs=[
            pl.BlockSpec((bn, hdim), lambda i: (i, 0)),
            pl.BlockSpec((bn, 128), lambda i: (i, 0)),
        ],
        out_shape=[
            jax.ShapeDtypeStruct((n, hdim), F32),
            jax.ShapeDtypeStruct((n, 128), F32),
        ],
    )(h, *([agg] * p), tsum, tsum, coord, wn1h, wn1a, bn1.reshape(1, -1),
      wn2, bn2.reshape(1, -1))


def _final_tc(h, w, b, bn):
    n, k = h.shape
    m = w.shape[1]
    nb = n // bn

    def body(h_ref, w_ref, b_ref, o_ref, acc_ref):
        i = pl.program_id(0)

        @pl.when(i == 0)
        def _():
            acc_ref[...] = jnp.zeros_like(acc_ref)

        acc_ref[...] += jnp.sum(h_ref[...], axis=0, keepdims=True)

        @pl.when(i == nb - 1)
        def _():
            o_ref[...] = (
                _bdot(acc_ref[...] / n, w_ref[...])
                + b_ref[...]
            )

    return pl.pallas_call(
        body,
        grid=(nb,),
        in_specs=[
            pl.BlockSpec((bn, k), lambda i: (i, 0)),
            pl.BlockSpec((k, m), lambda i: (0, 0)),
            pl.BlockSpec((1, m), lambda i: (0, 0)),
        ],
        out_specs=pl.BlockSpec((1, m), lambda i: (0, 0)),
        out_shape=jax.ShapeDtypeStruct((1, m), F32),
        scratch_shapes=[pltpu.VMEM((1, k), F32)],
    )(h, w, b.reshape(1, -1))


def _sc_gather(table, idx, ch):
    v, d = table.shape
    b = idx.shape[0]
    assert b % NW == 0
    bpw = b // NW
    assert bpw % ch == 0 and ch % 8 == 0
    nit = bpw // ch
    assert nit % 2 == 0 and nit >= 4
    mesh = plsc.VectorSubcoreMesh(core_axis_name="c", subcore_axis_name="s")

    @functools.partial(
        pl.kernel,
        mesh=mesh,
        out_type=jax.ShapeDtypeStruct((b, d), F32),
        scratch_types=[
            pltpu.VMEM((bpw,), jnp.int32),
            pltpu.VMEM((ch, d), F32),
            pltpu.VMEM((ch, d), F32),
            pltpu.SemaphoreType.DMA,
            pltpu.SemaphoreType.DMA,
            pltpu.SemaphoreType.DMA,
            pltpu.SemaphoreType.DMA,
        ],
    )
    def k(table_hbm, idx_hbm, out_hbm, idx_v, r0, r1, g0, g1, w0, w1):
        wid = lax.axis_index("s") * NC + lax.axis_index("c")
        base0 = pl.multiple_of(wid * bpw, 8)
        rows = (r0, r1)
        gs = (g0, g1)
        ws = (w0, w1)
        pltpu.sync_copy(idx_hbm.at[pl.ds(base0, bpw)], idx_v)
        for bb in range(2):
            pltpu.async_copy(
                table_hbm.at[idx_v.at[pl.ds(bb * ch, ch)]], rows[bb], gs[bb])

        def step(j, carry):
            for bb in range(2):
                i = 2 * j + bb
                pltpu.make_async_copy(
                    table_hbm.at[idx_v.at[pl.ds(0, ch)]], rows[bb],
                    gs[bb]).wait()
                off = pl.multiple_of(base0 + i * ch, 8)
                pltpu.async_copy(rows[bb], out_hbm.at[pl.ds(off, ch)], ws[bb])

                @pl.when(i + 2 < nit)
                def _():
                    pltpu.make_async_copy(
                        rows[bb], out_hbm.at[pl.ds(base0, ch)], ws[bb]).wait()
                    pltpu.async_copy(
                        table_hbm.at[idx_v.at[pl.ds((i + 2) * ch, ch)]],
                        rows[bb], gs[bb])

            return carry

        lax.fori_loop(0, nit // 2, step, 0)
        for bb in range(2):
            pltpu.make_async_copy(
                rows[bb], out_hbm.at[pl.ds(base0, ch)], ws[bb]).wait()

    return k(table, idx)


def _sc_scatter_cols(m, idx, zeros, n, c, bet):
    e, hdim = m.shape
    nck = hdim // c
    ncps = nck // NC
    ept = e // NS
    assert ept % bet == 0 and bet % 8 == 0
    nit = ept // bet
    rpt = (-(-n // NS) + 7) // 8 * 8
    tail = n - (NS - 1) * rpt
    assert tail > 0 and tail % 8 == 0
    mesh = plsc.VectorSubcoreMesh(core_axis_name="c", subcore_axis_name="s")

    @functools.partial(
        pl.kernel,
        mesh=mesh,
        out_type=jax.ShapeDtypeStruct((n, hdim), F32),
        scratch_types=[
            pltpu.VMEM((bet, c), F32),
            pltpu.VMEM((bet, c), F32),
            pltpu.VMEM((bet,), jnp.int32),
            pltpu.VMEM((bet,), jnp.int32),
            pltpu.VMEM_SHARED((n, c), F32),
            pltpu.SemaphoreType.DMA,
            pltpu.SemaphoreType.DMA,
            pltpu.SemaphoreType.DMA,
            pltpu.SemaphoreType.DMA,
        ],
    )
    def k(m_hbm, idx_hbm, z_hbm, out_hbm, m0, m1, i0, i1, acc,
          l0, l1, s0, s1):
        core = lax.axis_index("c")
        s = lax.axis_index("s")
        r0 = pl.multiple_of(s * rpt, 8)
        mb = (m0, m1)
        ib = (i0, i1)
        ls = (l0, l1)
        ss = (s0, s1)
        ebase = pl.multiple_of(s * ept, 8)

        def fire_load(i, bb, c0):
            e0 = pl.multiple_of(ebase + i * bet, 8)
            pltpu.async_copy(idx_hbm.at[pl.ds(e0, bet)], ib[bb], ls[bb])
            pltpu.async_copy(m_hbm.at[pl.ds(e0, bet), pl.ds(c0, c)],
                             mb[bb], ls[bb])

        for kk in range(ncps):
            chunk = core + NC * kk
            c0 = chunk * c

            @pl.when(s < NS - 1)
            def _():
                pltpu.sync_copy(z_hbm.at[pl.ds(0, rpt), pl.ds(0, c)],
                                acc.at[pl.ds(r0, rpt)])

            @pl.when(s == NS - 1)
            def _():
                pltpu.sync_copy(z_hbm.at[pl.ds(0, tail), pl.ds(0, c)],
                                acc.at[pl.ds(r0, tail)])

            plsc.subcore_barrier()
            for bb in range(2):
                fire_load(bb, bb, c0)

            def step(j, carry):
                for bb in range(2):
                    i = 2 * j + bb
                    pltpu.make_async_copy(
                        idx_hbm.at[pl.ds(0, bet)], ib[bb], ls[bb]).wait()
                    pltpu.make_async_copy(
                        m_hbm.at[pl.ds(0, bet), pl.ds(0, c)], mb[bb],
                        ls[bb]).wait()
                    pltpu.async_copy(mb[bb], acc.at[ib[bb]], ss[bb],
                                     add=True)

                    @pl.when(i + 2 < nit)
                    def _():
                        pltpu.make_async_copy(
                            mb[bb], acc.at[ib[bb]], ss[bb]).wait()
                        fire_load(i + 2, bb, c0)

                return carry

            lax.fori_loop(0, nit // 2, step, 0)
            if nit % 2 == 0:
                for bb in range(2):
                    pltpu.make_async_copy(mb[bb], acc.at[ib[bb]],
                                          ss[bb]).wait()
            else:
                pltpu.make_async_copy(mb[1], acc.at[ib[1]], ss[1]).wait()
                pltpu.make_async_copy(
                    idx_hbm.at[pl.ds(0, bet)], ib[0], ls[0]).wait()
                pltpu.make_async_copy(
                    m_hbm.at[pl.ds(0, bet), pl.ds(0, c)], mb[0],
                    ls[0]).wait()
                pltpu.async_copy(mb[0], acc.at[ib[0]], ss[0], add=True)
                pltpu.make_async_copy(mb[0], acc.at[ib[0]], ss[0]).wait()
            plsc.subcore_barrier()

            @pl.when(s < NS - 1)
            def _():
                pltpu.sync_copy(acc.at[pl.ds(r0, rpt)],
                                out_hbm.at[pl.ds(r0, rpt), pl.ds(c0, c)])

            @pl.when(s == NS - 1)
            def _():
                pltpu.sync_copy(acc.at[pl.ds(r0, tail)],
                                out_hbm.at[pl.ds(r0, tail), pl.ds(c0, c)])

            plsc.subcore_barrier()

    return k(m, idx, zeros)


def _sc_scatter_halves(m, idx, zeros, n, bet):
    e, c = m.shape
    eps = e // NC
    ept = eps // NS
    assert ept % bet == 0 and bet % 8 == 0
    nit = ept // bet
    rpt = (-(-n // NS) + 7) // 8 * 8
    tail = n - (NS - 1) * rpt
    assert tail > 0 and tail % 8 == 0
    mesh = plsc.VectorSubcoreMesh(core_axis_name="c", subcore_axis_name="s")

    assert nit >= 4
    npipe = nit // 2 * 2

    @functools.partial(
        pl.kernel,
        mesh=mesh,
        out_type=jax.ShapeDtypeStruct((2 * n, c), F32),
        scratch_types=[
            pltpu.VMEM((bet, c), F32),
            pltpu.VMEM((bet, c), F32),
            pltpu.VMEM((bet,), jnp.int32),
            pltpu.VMEM((bet,), jnp.int32),
            pltpu.VMEM_SHARED((n, c), F32),
            pltpu.SemaphoreType.DMA,
            pltpu.SemaphoreType.DMA,
            pltpu.SemaphoreType.DMA,
            pltpu.SemaphoreType.DMA,
        ],
    )
    def k(m_hbm, idx_hbm, z_hbm, out_hbm, m0, m1, i0, i1, acc,
          l0, l1, s0, s1):
        core = lax.axis_index("c")
        s = lax.axis_index("s")
        r0 = pl.multiple_of(s * rpt, 8)
        o0 = pl.multiple_of(core * n + s * rpt, 8)
        mb = (m0, m1)
        ib = (i0, i1)
        ls = (l0, l1)
        ss = (s0, s1)
        ebase = pl.multiple_of(core * eps + s * ept, 8)

        def fire_load(i, bb):
            e0 = pl.multiple_of(ebase + i * bet, 8)
            pltpu.async_copy(idx_hbm.at[pl.ds(e0, bet)], ib[bb], ls[bb])
            pltpu.async_copy(m_hbm.at[pl.ds(e0, bet), pl.ds(0, c)],
                             mb[bb], ls[bb])

        def wait_load(bb):
            pltpu.make_async_copy(
                idx_hbm.at[pl.ds(0, bet)], ib[bb], ls[bb]).wait()
            pltpu.make_async_copy(
                m_hbm.at[pl.ds(0, bet), pl.ds(0, c)], mb[bb], ls[bb]).wait()

        @pl.when(s < NS - 1)
        def _():
            pltpu.sync_copy(z_hbm.at[pl.ds(0, rpt), pl.ds(0, c)],
                            acc.at[pl.ds(r0, rpt)])

        @pl.when(s == NS - 1)
        def _():
            pltpu.sync_copy(z_hbm.at[pl.ds(0, tail), pl.ds(0, c)],
                            acc.at[pl.ds(r0, tail)])

        plsc.subcore_barrier()
        for bb in range(2):
            fire_load(bb, bb)

        def step(j, carry):
            for bb in range(2):
                i = 2 * j + bb
                wait_load(bb)
                pltpu.async_copy(mb[bb], acc.at[ib[bb]], ss[bb], add=True)

                @pl.when(i + 2 < nit)
                def _():
                    pltpu.make_async_copy(mb[bb], acc.at[ib[bb]],
                                          ss[bb]).wait()
                    fire_load(i + 2, bb)

            return carry

        lax.fori_loop(0, nit // 2, step, 0)
        if npipe == nit:
            for bb in range(2):
                pltpu.make_async_copy(mb[bb], acc.at[ib[bb]], ss[bb]).wait()
        else:
            pltpu.make_async_copy(mb[1], acc.at[ib[1]], ss[1]).wait()
            wait_load(0)
            pltpu.async_copy(mb[0], acc.at[ib[0]], ss[0], add=True)
            pltpu.make_async_copy(mb[0], acc.at[ib[0]], ss[0]).wait()
        plsc.subcore_barrier()

        @pl.when(s < NS - 1)
        def _():
            pltpu.sync_copy(acc.at[pl.ds(r0, rpt)],
                            out_hbm.at[pl.ds(o0, rpt), pl.ds(0, c)])

        @pl.when(s == NS - 1)
        def _():
            pltpu.sync_copy(acc.at[pl.ds(r0, tail)],
                            out_hbm.at[pl.ds(o0, tail), pl.ds(0, c)])

    return k(m, idx, zeros)


def _egcl(h, coord, lp, idx_pq, row, ea16, zeros, n, e, hdim,
          bn, be, ch_pq, bet_m, cols_mode):
    w1 = lp["edge1"]["w"]
    wa, wb = w1[:hdim], w1[hdim:2 * hdim]
    wr = w1[2 * hdim:2 * hdim + 1]
    we = jnp.pad(w1[2 * hdim + 1:], ((0, 5), (0, 0)))
    b1 = lp["edge1"]["b"]

    pq = _pq_tc(h, jnp.stack([wa, wb]),
                jnp.stack([b1.reshape(1, -1), jnp.zeros((1, hdim), F32)]),
                coord, bn)
    g2 = _sc_gather(pq, idx_pq, ch_pq)

    m, trans = _edge_tc(
        g2, ea16, wr, we, lp["edge2"]["w"], lp["edge2"]["b"].reshape(1, -1),
        lp["att"]["w"].reshape(1, -1), lp["att"]["b"].reshape(1, 1),
        lp["coord1"]["w"], lp["coord1"]["b"].reshape(1, -1),
        lp["coord2_w"].reshape(1, -1), e, hdim, be)

    if cols_mode:
        agg = _sc_scatter_cols(m, row, zeros, n, 128, bet_m)
    else:
        agg = _sc_scatter_halves(m, row, zeros, n, bet_m)
    tsum = _sc_scatter_halves(trans, row, zeros, n, 40)

    wn1 = lp["node1"]["w"]
    h, coord = _node_tc(h, agg, tsum, coord, wn1[:hdim], wn1[hdim:],
                        lp["node1"]["b"], lp["node2"]["w"], lp["node2"]["b"],
                        bn)
    return h, coord


def kernel(node_features, edge_index, node_pos, edge_attr, params):
    n = node_features.shape[0]
    e = edge_index.shape[1]
    row = edge_index[0]
    col = edge_index[1]
    idx_pq = jnp.concatenate([row, col + n])
    ea16 = jnp.pad(edge_attr, ((0, 0), (0, 16 - edge_attr.shape[1])))
    zeros = jnp.zeros((n, 128), F32)
    coord = jnp.pad(node_pos, ((0, 0), (0, 125)))
    bn = 1000

    p1 = params["egnn1"]
    h = _linear_tc(node_features, p1["emb_in"]["w"], p1["emb_in"]["b"], bn)
    for lp in p1["layers"]:
        h, coord = _egcl(h, coord, lp, idx_pq, row, ea16, zeros,
                         n, e, 512, bn, 640, 40, 80, cols_mode=True)
    h = _linear_tc(h, p1["emb_out"]["w"], p1["emb_out"]["b"], bn)

    p2 = params["egnn2"]
    h = _linear_tc(h, p2["emb_in"]["w"], p2["emb_in"]["b"], bn)
    for lp in p2["layers"]:
        h, coord = _egcl(h, coord, lp, idx_pq, row, ea16, zeros,
                         n, e, 128, bn, 2000, 200, 40, cols_mode=False)
    out = _final_tc(h, p2["emb_out"]["w"], p2["emb_out"]["b"], bn)
    return out


# speedup vs baseline: 2.7709x; 1.0914x over previous
import functools

import jax
import jax.numpy as jnp
from jax import lax
from jax.experimental import pallas as pl
from jax.experimental.pallas import tpu as pltpu
from jax.experimental.pallas import tpu_sc as plsc

F32 = jnp.float32
BF16 = jnp.bfloat16
PREC = jax.lax.Precision.HIGHEST


def _bdot(a, b):
    return jnp.dot(a.astype(BF16), b.astype(BF16),
                   preferred_element_type=F32)


def _br(x):
    return x.astype(BF16).astype(F32)
NC, NS = 2, 16
NW = NC * NS


def _silu(x):
    return x * jax.nn.sigmoid(x)


def _linear_tc(h, w, b, bn):
    n, k = h.shape
    m = w.shape[1]
    nb = n // bn

    def body(h_ref, w_ref, b_ref, o_ref):
        o_ref[...] = (
            _bdot(h_ref[...], w_ref[...])
            + b_ref[...]
        )

    return pl.pallas_call(
        body,
        grid=(nb,),
        in_specs=[
            pl.BlockSpec((bn, k), lambda i: (i, 0)),
            pl.BlockSpec((k, m), lambda i: (0, 0)),
            pl.BlockSpec((1, m), lambda i: (0, 0)),
        ],
        out_specs=pl.BlockSpec((bn, m), lambda i: (i, 0)),
        out_shape=jax.ShapeDtypeStruct((n, m), F32),
    )(h, w, b.reshape(1, -1))


def _pq_tc(h, wab, bab, coord, bn):
    n, k = h.shape
    m = wab.shape[2]
    nb = n // bn

    packed = m >= 256
    pk = m // 2 if packed else m
    width = pk + 128

    def body(h_ref, w_ref, b_ref, c_ref, o_ref):
        pq = _bdot(h_ref[...], w_ref[0]) + b_ref[0]
        if packed:
            u = lax.bitcast_convert_type(pq.astype(BF16), jnp.uint16)
            wds = ((u[:, :pk].astype(jnp.uint32) << 16)
                   | u[:, pk:].astype(jnp.uint32))
            o_ref[:, :pk] = lax.bitcast_convert_type(wds, F32)
        else:
            o_ref[:, :pk] = pq
        o_ref[:, pk:] = c_ref[...]

    return pl.pallas_call(
        body,
        grid=(2, nb),
        in_specs=[
            pl.BlockSpec((bn, k), lambda j, i: (i, 0)),
            pl.BlockSpec((1, k, m), lambda j, i: (j, 0, 0)),
            pl.BlockSpec((1, 1, m), lambda j, i: (j, 0, 0)),
            pl.BlockSpec((bn, 128), lambda j, i: (i, 0)),
        ],
        out_specs=pl.BlockSpec((bn, width), lambda j, i: (j * nb + i, 0)),
        out_shape=jax.ShapeDtypeStruct((2 * n, width), F32),
    )(h, wab, bab, coord)


def _edge_tc(g2, ea, wr, we, w2, b2, watt, batt, wc1, bc1, wc2, e, hdim, be):
    nbe = e // be
    packed = hdim >= 256
    pk = hdim // 2 if packed else hdim
    w = pk + 128

    def body(gr_ref, gc_ref, ea_ref, wr_ref, we_ref, w2_ref,
             b2_ref, watt_ref, batt_ref, wc1_ref, bc1_ref, wc2_ref,
             m_ref, t_ref):
        def unpack(ref):
            if not packed:
                return ref[:, :pk], ref[:, pk:]
            u = lax.bitcast_convert_type(ref[:, :pk], jnp.uint32)
            hi = lax.bitcast_convert_type(
                (u >> 16).astype(jnp.uint16), BF16).astype(F32)
            lo = lax.bitcast_convert_type(
                (u & 0xFFFF).astype(jnp.uint16), BF16).astype(F32)
            return jnp.concatenate([hi, lo], axis=1), ref[:, pk:]

        gr, xr = unpack(gr_ref)
        gc, xc = unpack(gc_ref)
        diff = xr - xc
        radial = jnp.sum(diff * diff, axis=1, keepdims=True)
        dn = diff / (jnp.sqrt(radial) + 1e-8)
        t = (gr + gc
             + _br(radial) * _br(wr_ref[...])
             + _bdot(ea_ref[...], we_ref[...]))
        m1 = _silu(t)
        m2 = _bdot(m1, w2_ref[...]) + b2_ref[...]
        m = _silu(m2)
        attl = (jnp.sum(_br(m) * _br(watt_ref[...]), axis=1, keepdims=True)
                + batt_ref[...])
        m = m * jax.nn.sigmoid(attl)
        cm = _bdot(m, wc1_ref[...]) + bc1_ref[...]
        cm = _silu(cm)
        s = jnp.sum(_br(cm) * _br(wc2_ref[...]), axis=1, keepdims=True)
        e3 = (lax.broadcasted_iota(jnp.int32, (1, 128), 1) == 3).astype(F32)
        m_ref[...] = m
        t_ref[...] = dn * s + e3

    cw = lambda shp: pl.BlockSpec(shp, lambda i: tuple(0 for _ in shp))
    return pl.pallas_call(
        body,
        grid=(nbe,),
        in_specs=[
            pl.BlockSpec((be, w), lambda i: (i, 0)),
            pl.BlockSpec((be, w), lambda i: (nbe + i, 0)),
            pl.BlockSpec((be, 16), lambda i: (i, 0)),
            cw((1, hdim)), cw((16, hdim)), cw((hdim, hdim)), cw((1, hdim)),
            cw((1, hdim)), cw((1, 1)), cw((hdim, hdim)), cw((1, hdim)),
            cw((1, hdim)),
        ],
        out_specs=[
            pl.BlockSpec((be, hdim), lambda i: (i, 0)),
            pl.BlockSpec((be, 128), lambda i: (i, 0)),
        ],
        out_shape=[
            jax.ShapeDtypeStruct((e, hdim), F32),
            jax.ShapeDtypeStruct((e, 128), F32),
        ],
    )(g2, g2, ea, wr, we, w2, b2, watt, batt, wc1, bc1, wc2)


def _node_tc(h, agg, tsum, coord, wn1h, wn1a, bn1, wn2, bn2, bn):
    n, hdim = h.shape
    p = agg.shape[0] // n
    nb = n // bn

    def body(*refs):
        h_ref = refs[0]
        aggs = refs[1:1 + p]
        t0, t1 = refs[1 + p], refs[2 + p]
        coord_ref = refs[3 + p]
        wn1h_ref, wn1a_ref, bn1_ref, wn2_ref, bn2_ref = refs[4 + p:9 + p]
        h_out, c_out = refs[9 + p], refs[10 + p]

        a = aggs[0][...]
        if p == 2:
            a = a + aggs[1][...]
        ts = t0[...] + t1[...]
        lane = lax.broadcasted_iota(jnp.int32, (1, 128), 1)
        mask3 = (lane < 3).astype(F32)
        e3 = (lane == 3).astype(F32)
        cnt = jnp.sum(ts * e3, axis=1, keepdims=True)
        inv = 1.0 / jnp.maximum(cnt, 1.0)
        c_out[...] = coord_ref[...] + ts * mask3 * inv
        tmp = (_bdot(h_ref[...], wn1h_ref[...])
               + _bdot(a, wn1a_ref[...])
               + bn1_ref[...])
        tmp = _silu(tmp)
        h_out[...] = (h_ref[...]
                      + _bdot(tmp, wn2_ref[...])
                      + bn2_ref[...])

    cw = lambda shp: pl.BlockSpec(shp, lambda i: tuple(0 for _ in shp))
    in_specs = [pl.BlockSpec((bn, hdim), lambda i: (i, 0))]
    for q in range(p):
        in_specs.append(pl.BlockSpec((bn, hdim), lambda i, q=q: (q * nb + i, 0)))
    in_specs += [
        pl.BlockSpec((bn, 128), lambda i: (i, 0)),
        pl.BlockSpec((bn, 128), lambda i: (nb + i, 0)),
        pl.BlockSpec((bn, 128), lambda i: (i, 0)),
        cw((hdim, hdim)), cw((hdim, hdim)), cw((1, hdim)),
        cw((hdim, hdim)), cw((1, hdim)),
    ]
    return pl.pallas_call(
        body,
        grid=(nb,),
        in_specs=in_specs,
        out_specs=[
            pl.BlockSpec((bn, hdim), lambda i: (i, 0)),
            pl.BlockSpec((bn, 128), lambda i: (i, 0)),
        ],
        out_shape=[
            jax.ShapeDtypeStruct((n, hdim), F32),
            jax.ShapeDtypeStruct((n, 128), F32),
        ],
    )(h, *([agg] * p), tsum, tsum, coord, wn1h, wn1a, bn1.reshape(1, -1),
      wn2, bn2.reshape(1, -1))


def _final_tc(h, w, b, bn):
    n, k = h.shape
    m = w.shape[1]
    nb = n // bn

    def body(h_ref, w_ref, b_ref, o_ref, acc_ref):
        i = pl.program_id(0)

        @pl.when(i == 0)
        def _():
            acc_ref[...] = jnp.zeros_like(acc_ref)

        acc_ref[...] += jnp.sum(h_ref[...], axis=0, keepdims=True)

        @pl.when(i == nb - 1)
        def _():
            o_ref[...] = (
                _bdot(acc_ref[...] / n, w_ref[...])
                + b_ref[...]
            )

    return pl.pallas_call(
        body,
        grid=(nb,),
        in_specs=[
            pl.BlockSpec((bn, k), lambda i: (i, 0)),
            pl.BlockSpec((k, m), lambda i: (0, 0)),
            pl.BlockSpec((1, m), lambda i: (0, 0)),
        ],
        out_specs=pl.BlockSpec((1, m), lambda i: (0, 0)),
        out_shape=jax.ShapeDtypeStruct((1, m), F32),
        scratch_shapes=[pltpu.VMEM((1, k), F32)],
    )(h, w, b.reshape(1, -1))


def _sc_gather(table, idx, ch):
    v, d = table.shape
    dt = table.dtype
    b = idx.shape[0]
    assert b % NW == 0
    bpw = b // NW
    assert bpw % ch == 0 and ch % 16 == 0
    nit = bpw // ch
    assert nit >= 4
    mesh = plsc.VectorSubcoreMesh(core_axis_name="c", subcore_axis_name="s")

    @functools.partial(
        pl.kernel,
        mesh=mesh,
        out_type=jax.ShapeDtypeStruct((b, d), dt),
        scratch_types=[
            pltpu.VMEM((bpw,), jnp.int32),
            pltpu.VMEM((ch, d), dt),
            pltpu.VMEM((ch, d), dt),
            pltpu.SemaphoreType.DMA,
            pltpu.SemaphoreType.DMA,
            pltpu.SemaphoreType.DMA,
            pltpu.SemaphoreType.DMA,
        ],
    )
    def k(table_hbm, idx_hbm, out_hbm, idx_v, r0, r1, g0, g1, w0, w1):
        wid = lax.axis_index("s") * NC + lax.axis_index("c")
        base0 = pl.multiple_of(wid * bpw, 8)
        rows = (r0, r1)
        gs = (g0, g1)
        ws = (w0, w1)
        pltpu.sync_copy(idx_hbm.at[pl.ds(base0, bpw)], idx_v)
        for bb in range(2):
            pltpu.async_copy(
                table_hbm.at[idx_v.at[pl.ds(bb * ch, ch)]], rows[bb], gs[bb])

        def step(j, carry):
            for bb in range(2):
                i = 2 * j + bb
                pltpu.make_async_copy(
                    table_hbm.at[idx_v.at[pl.ds(0, ch)]], rows[bb],
                    gs[bb]).wait()
                off = pl.multiple_of(base0 + i * ch, 8)
                pltpu.async_copy(rows[bb], out_hbm.at[pl.ds(off, ch)], ws[bb])

                @pl.when(i + 2 < nit)
                def _():
                    pltpu.make_async_copy(
                        rows[bb], out_hbm.at[pl.ds(base0, ch)], ws[bb]).wait()
                    pltpu.async_copy(
                        table_hbm.at[idx_v.at[pl.ds((i + 2) * ch, ch)]],
                        rows[bb], gs[bb])

            return carry

        lax.fori_loop(0, nit // 2, step, 0)
        if nit % 2 == 1:
            pltpu.make_async_copy(
                table_hbm.at[idx_v.at[pl.ds(0, ch)]], rows[0], gs[0]).wait()
            off = pl.multiple_of(base0 + (nit - 1) * ch, 8)
            pltpu.async_copy(rows[0], out_hbm.at[pl.ds(off, ch)], ws[0])
        for bb in range(2):
            pltpu.make_async_copy(
                rows[bb], out_hbm.at[pl.ds(base0, ch)], ws[bb]).wait()

    return k(table, idx)


def _sc_scatter_cols(m, idx, zeros, n, c, bet):
    e, hdim = m.shape
    nck = hdim // c
    ncps = nck // NC
    ept = e // NS
    assert ept % bet == 0 and bet % 8 == 0
    nit = ept // bet
    rpt = (-(-n // NS) + 7) // 8 * 8
    tail = n - (NS - 1) * rpt
    assert tail > 0 and tail % 8 == 0
    mesh = plsc.VectorSubcoreMesh(core_axis_name="c", subcore_axis_name="s")

    @functools.partial(
        pl.kernel,
        mesh=mesh,
        out_type=jax.ShapeDtypeStruct((n, hdim), F32),
        scratch_types=[
            pltpu.VMEM((bet, c), F32),
            pltpu.VMEM((bet, c), F32),
            pltpu.VMEM((bet,), jnp.int32),
            pltpu.VMEM((bet,), jnp.int32),
            pltpu.VMEM_SHARED((n, c), F32),
            pltpu.SemaphoreType.DMA,
            pltpu.SemaphoreType.DMA,
            pltpu.SemaphoreType.DMA,
            pltpu.SemaphoreType.DMA,
        ],
    )
    def k(m_hbm, idx_hbm, z_hbm, out_hbm, m0, m1, i0, i1, acc,
          l0, l1, s0, s1):
        core = lax.axis_index("c")
        s = lax.axis_index("s")
        r0 = pl.multiple_of(s * rpt, 8)
        mb = (m0, m1)
        ib = (i0, i1)
        ls = (l0, l1)
        ss = (s0, s1)
        ebase = pl.multiple_of(s * ept, 8)

        def fire_load(i, bb, c0):
            e0 = pl.multiple_of(ebase + i * bet, 8)
            pltpu.async_copy(idx_hbm.at[pl.ds(e0, bet)], ib[bb], ls[bb])
            pltpu.async_copy(m_hbm.at[pl.ds(e0, bet), pl.ds(c0, c)],
                             mb[bb], ls[bb])

        for kk in range(ncps):
            chunk = core + NC * kk
            c0 = chunk * c

            @pl.when(s < NS - 1)
            def _():
                pltpu.sync_copy(z_hbm.at[pl.ds(0, rpt), pl.ds(0, c)],
                                acc.at[pl.ds(r0, rpt)])

            @pl.when(s == NS - 1)
            def _():
                pltpu.sync_copy(z_hbm.at[pl.ds(0, tail), pl.ds(0, c)],
                                acc.at[pl.ds(r0, tail)])

            plsc.subcore_barrier()
            for bb in range(2):
                fire_load(bb, bb, c0)

            def step(j, carry):
                for bb in range(2):
                    i = 2 * j + bb
                    pltpu.make_async_copy(
                        idx_hbm.at[pl.ds(0, bet)], ib[bb], ls[bb]).wait()
                    pltpu.make_async_copy(
                        m_hbm.at[pl.ds(0, bet), pl.ds(0, c)], mb[bb],
                        ls[bb]).wait()
                    pltpu.async_copy(mb[bb], acc.at[ib[bb]], ss[bb],
                                     add=True)

                    @pl.when(i + 2 < nit)
                    def _():
                        pltpu.make_async_copy(
                            mb[bb], acc.at[ib[bb]], ss[bb]).wait()
                        fire_load(i + 2, bb, c0)

                return carry

            lax.fori_loop(0, nit // 2, step, 0)
            if nit % 2 == 0:
                for bb in range(2):
                    pltpu.make_async_copy(mb[bb], acc.at[ib[bb]],
                                          ss[bb]).wait()
            else:
                pltpu.make_async_copy(mb[1], acc.at[ib[1]], ss[1]).wait()
                pltpu.make_async_copy(
                    idx_hbm.at[pl.ds(0, bet)], ib[0], ls[0]).wait()
                pltpu.make_async_copy(
                    m_hbm.at[pl.ds(0, bet), pl.ds(0, c)], mb[0],
                    ls[0]).wait()
                pltpu.async_copy(mb[0], acc.at[ib[0]], ss[0], add=True)
                pltpu.make_async_copy(mb[0], acc.at[ib[0]], ss[0]).wait()
            plsc.subcore_barrier()

            @pl.when(s < NS - 1)
            def _():
                pltpu.sync_copy(acc.at[pl.ds(r0, rpt)],
                                out_hbm.at[pl.ds(r0, rpt), pl.ds(c0, c)])

            @pl.when(s == NS - 1)
            def _():
                pltpu.sync_copy(acc.at[pl.ds(r0, tail)],
                                out_hbm.at[pl.ds(r0, tail), pl.ds(c0, c)])

            plsc.subcore_barrier()

    return k(m, idx, zeros)


def _sc_scatter_halves(m, idx, zeros, n, bet):
    e, c = m.shape
    eps = e // NC
    ept = eps // NS
    assert ept % bet == 0 and bet % 8 == 0
    nit = ept // bet
    rpt = (-(-n // NS) + 7) // 8 * 8
    tail = n - (NS - 1) * rpt
    assert tail > 0 and tail % 8 == 0
    mesh = plsc.VectorSubcoreMesh(core_axis_name="c", subcore_axis_name="s")

    assert nit >= 4
    npipe = nit // 2 * 2

    @functools.partial(
        pl.kernel,
        mesh=mesh,
        out_type=jax.ShapeDtypeStruct((2 * n, c), F32),
        scratch_types=[
            pltpu.VMEM((bet, c), F32),
            pltpu.VMEM((bet, c), F32),
            pltpu.VMEM((bet,), jnp.int32),
            pltpu.VMEM((bet,), jnp.int32),
            pltpu.VMEM_SHARED((n, c), F32),
            pltpu.SemaphoreType.DMA,
            pltpu.SemaphoreType.DMA,
            pltpu.SemaphoreType.DMA,
            pltpu.SemaphoreType.DMA,
        ],
    )
    def k(m_hbm, idx_hbm, z_hbm, out_hbm, m0, m1, i0, i1, acc,
          l0, l1, s0, s1):
        core = lax.axis_index("c")
        s = lax.axis_index("s")
        r0 = pl.multiple_of(s * rpt, 8)
        o0 = pl.multiple_of(core * n + s * rpt, 8)
        mb = (m0, m1)
        ib = (i0, i1)
        ls = (l0, l1)
        ss = (s0, s1)
        ebase = pl.multiple_of(core * eps + s * ept, 8)

        def fire_load(i, bb):
            e0 = pl.multiple_of(ebase + i * bet, 8)
            pltpu.async_copy(idx_hbm.at[pl.ds(e0, bet)], ib[bb], ls[bb])
            pltpu.async_copy(m_hbm.at[pl.ds(e0, bet), pl.ds(0, c)],
                             mb[bb], ls[bb])

        def wait_load(bb):
            pltpu.make_async_copy(
                idx_hbm.at[pl.ds(0, bet)], ib[bb], ls[bb]).wait()
            pltpu.make_async_copy(
                m_hbm.at[pl.ds(0, bet), pl.ds(0, c)], mb[bb], ls[bb]).wait()

        @pl.when(s < NS - 1)
        def _():
            pltpu.sync_copy(z_hbm.at[pl.ds(0, rpt), pl.ds(0, c)],
                            acc.at[pl.ds(r0, rpt)])

        @pl.when(s == NS - 1)
        def _():
            pltpu.sync_copy(z_hbm.at[pl.ds(0, tail), pl.ds(0, c)],
                            acc.at[pl.ds(r0, tail)])

        plsc.subcore_barrier()
        for bb in range(2):
            fire_load(bb, bb)

        def step(j, carry):
            for bb in range(2):
                i = 2 * j + bb
                wait_load(bb)
                pltpu.async_copy(mb[bb], acc.at[ib[bb]], ss[bb], add=True)

                @pl.when(i + 2 < nit)
                def _():
                    pltpu.make_async_copy(mb[bb], acc.at[ib[bb]],
                                          ss[bb]).wait()
                    fire_load(i + 2, bb)

            return carry

        lax.fori_loop(0, nit // 2, step, 0)
        if npipe == nit:
            for bb in range(2):
                pltpu.make_async_copy(mb[bb], acc.at[ib[bb]], ss[bb]).wait()
        else:
            pltpu.make_async_copy(mb[1], acc.at[ib[1]], ss[1]).wait()
            wait_load(0)
            pltpu.async_copy(mb[0], acc.at[ib[0]], ss[0], add=True)
            pltpu.make_async_copy(mb[0], acc.at[ib[0]], ss[0]).wait()
        plsc.subcore_barrier()

        @pl.when(s < NS - 1)
        def _():
            pltpu.sync_copy(acc.at[pl.ds(r0, rpt)],
                            out_hbm.at[pl.ds(o0, rpt), pl.ds(0, c)])

        @pl.when(s == NS - 1)
        def _():
            pltpu.sync_copy(acc.at[pl.ds(r0, tail)],
                            out_hbm.at[pl.ds(o0, tail), pl.ds(0, c)])

    return k(m, idx, zeros)


def _egcl(h, coord, lp, idx_pq, row, ea16, zeros, n, e, hdim,
          bn, be, ch_pq, bet_m, cols_mode):
    w1 = lp["edge1"]["w"]
    wa, wb = w1[:hdim], w1[hdim:2 * hdim]
    wr = w1[2 * hdim:2 * hdim + 1]
    we = jnp.pad(w1[2 * hdim + 1:], ((0, 5), (0, 0)))
    b1 = lp["edge1"]["b"]

    pq = _pq_tc(h, jnp.stack([wa, wb]),
                jnp.stack([b1.reshape(1, -1), jnp.zeros((1, hdim), F32)]),
                coord, bn)
    g2 = _sc_gather(pq, idx_pq, ch_pq)

    m, trans = _edge_tc(
        g2, ea16, wr, we, lp["edge2"]["w"], lp["edge2"]["b"].reshape(1, -1),
        lp["att"]["w"].reshape(1, -1), lp["att"]["b"].reshape(1, 1),
        lp["coord1"]["w"], lp["coord1"]["b"].reshape(1, -1),
        lp["coord2_w"].reshape(1, -1), e, hdim, be)

    if cols_mode:
        agg = _sc_scatter_cols(m, row, zeros, n, 128, bet_m)
    else:
        agg = _sc_scatter_halves(m, row, zeros, n, bet_m)
    tsum = _sc_scatter_halves(trans, row, zeros, n, 40)

    wn1 = lp["node1"]["w"]
    h, coord = _node_tc(h, agg, tsum, coord, wn1[:hdim], wn1[hdim:],
                        lp["node1"]["b"], lp["node2"]["w"], lp["node2"]["b"],
                        bn)
    return h, coord


def kernel(node_features, edge_index, node_pos, edge_attr, params):
    n = node_features.shape[0]
    e = edge_index.shape[1]
    row = edge_index[0]
    col = edge_index[1]
    idx_pq = jnp.concatenate([row, col + n])
    ea16 = jnp.pad(edge_attr, ((0, 0), (0, 16 - edge_attr.shape[1])))
    zeros = jnp.zeros((n, 128), F32)
    coord = jnp.pad(node_pos, ((0, 0), (0, 125)))
    bn = 1000

    p1 = params["egnn1"]
    h = _linear_tc(node_features, p1["emb_in"]["w"], p1["emb_in"]["b"], bn)
    for lp in p1["layers"]:
        h, coord = _egcl(h, coord, lp, idx_pq, row, ea16, zeros,
                         n, e, 512, bn, 640, 80, 80, cols_mode=True)
    h = _linear_tc(h, p1["emb_out"]["w"], p1["emb_out"]["b"], bn)

    p2 = params["egnn2"]
    h = _linear_tc(h, p2["emb_in"]["w"], p2["emb_in"]["b"], bn)
    for lp in p2["layers"]:
        h, coord = _egcl(h, coord, lp, idx_pq, row, ea16, zeros,
                         n, e, 128, bn, 2000, 80, 40, cols_mode=False)
    out = _final_tc(h, p2["emb_out"]["w"], p2["emb_out"]["b"], bn)
    return out


# speedup vs baseline: 2.9647x; 1.0699x over previous
import functools

import jax
import jax.numpy as jnp
from jax import lax
from jax.experimental import pallas as pl
from jax.experimental.pallas import tpu as pltpu
from jax.experimental.pallas import tpu_sc as plsc

F32 = jnp.float32
BF16 = jnp.bfloat16
PREC = jax.lax.Precision.HIGHEST


def _bdot(a, b):
    return jnp.dot(a.astype(BF16), b.astype(BF16),
                   preferred_element_type=F32)


def _br(x):
    return x.astype(BF16).astype(F32)
NC, NS = 2, 16
NW = NC * NS


def _silu(x):
    return x * jax.nn.sigmoid(x)


def _linear_tc(h, w, b, bn):
    n, k = h.shape
    m = w.shape[1]
    nb = n // bn

    def body(h_ref, w_ref, b_ref, o_ref):
        o_ref[...] = (
            _bdot(h_ref[...], w_ref[...])
            + b_ref[...]
        )

    return pl.pallas_call(
        body,
        grid=(nb,),
        in_specs=[
            pl.BlockSpec((bn, k), lambda i: (i, 0)),
            pl.BlockSpec((k, m), lambda i: (0, 0)),
            pl.BlockSpec((1, m), lambda i: (0, 0)),
        ],
        out_specs=pl.BlockSpec((bn, m), lambda i: (i, 0)),
        out_shape=jax.ShapeDtypeStruct((n, m), F32),
    )(h, w, b.reshape(1, -1))


def _pq_tc(h, wab, bab, coord, bn):
    n, k = h.shape
    m = wab.shape[2]
    nb = n // bn

    packed = m >= 256
    pk = m // 2 if packed else m
    width = pk + 128

    def body(h_ref, w_ref, b_ref, c_ref, o_ref):
        pq = _bdot(h_ref[...], w_ref[0]) + b_ref[0]
        if packed:
            u = lax.bitcast_convert_type(pq.astype(BF16), jnp.uint16)
            wds = ((u[:, :pk].astype(jnp.uint32) << 16)
                   | u[:, pk:].astype(jnp.uint32))
            o_ref[:, :pk] = lax.bitcast_convert_type(wds, F32)
        else:
            o_ref[:, :pk] = pq
        o_ref[:, pk:] = c_ref[...]

    return pl.pallas_call(
        body,
        grid=(2, nb),
        in_specs=[
            pl.BlockSpec((bn, k), lambda j, i: (i, 0)),
            pl.BlockSpec((1, k, m), lambda j, i: (j, 0, 0)),
            pl.BlockSpec((1, 1, m), lambda j, i: (j, 0, 0)),
            pl.BlockSpec((bn, 128), lambda j, i: (i, 0)),
        ],
        out_specs=pl.BlockSpec((bn, width), lambda j, i: (j * nb + i, 0)),
        out_shape=jax.ShapeDtypeStruct((2 * n, width), F32),
    )(h, wab, bab, coord)


def _edge_tc(g2, ea, wr, we, w2, b2, watt, batt, wc1, bc1, wc2, e, hdim, be):
    nbe = e // be
    packed = hdim >= 256
    pk = hdim // 2 if packed else hdim
    w = pk + 128

    def body(gr_ref, gc_ref, ea_ref, wr_ref, we_ref, w2_ref,
             b2_ref, watt_ref, batt_ref, wc1_ref, bc1_ref, wc2_ref,
             m_ref, t_ref):
        def unpack(ref):
            if not packed:
                return ref[:, :pk], ref[:, pk:]
            u = lax.bitcast_convert_type(ref[:, :pk], jnp.uint32)
            hi = lax.bitcast_convert_type(
                (u >> 16).astype(jnp.uint16), BF16).astype(F32)
            lo = lax.bitcast_convert_type(
                (u & 0xFFFF).astype(jnp.uint16), BF16).astype(F32)
            return jnp.concatenate([hi, lo], axis=1), ref[:, pk:]

        gr, xr = unpack(gr_ref)
        gc, xc = unpack(gc_ref)
        diff = xr - xc
        radial = jnp.sum(diff * diff, axis=1, keepdims=True)
        dn = diff / (jnp.sqrt(radial) + 1e-8)
        t = (gr + gc
             + _br(radial) * _br(wr_ref[...])
             + _bdot(ea_ref[...], we_ref[...]))
        m1 = _silu(t)
        m2 = _bdot(m1, w2_ref[...]) + b2_ref[...]
        m = _silu(m2)
        attl = (jnp.sum(_br(m) * _br(watt_ref[...]), axis=1, keepdims=True)
                + batt_ref[...])
        m = m * jax.nn.sigmoid(attl)
        cm = _bdot(m, wc1_ref[...]) + bc1_ref[...]
        cm = _silu(cm)
        s = jnp.sum(_br(cm) * _br(wc2_ref[...]), axis=1, keepdims=True)
        e3 = (lax.broadcasted_iota(jnp.int32, (1, 128), 1) == 3).astype(F32)
        m_ref[...] = m
        t_ref[...] = dn * s + e3

    cw = lambda shp: pl.BlockSpec(shp, lambda i: tuple(0 for _ in shp))
    return pl.pallas_call(
        body,
        grid=(nbe,),
        in_specs=[
            pl.BlockSpec((be, w), lambda i: (i, 0)),
            pl.BlockSpec((be, w), lambda i: (nbe + i, 0)),
            pl.BlockSpec((be, 16), lambda i: (i, 0)),
            cw((1, hdim)), cw((16, hdim)), cw((hdim, hdim)), cw((1, hdim)),
            cw((1, hdim)), cw((1, 1)), cw((hdim, hdim)), cw((1, hdim)),
            cw((1, hdim)),
        ],
        out_specs=[
            pl.BlockSpec((be, hdim), lambda i: (i, 0)),
            pl.BlockSpec((be, 128), lambda i: (i, 0)),
        ],
        out_shape=[
            jax.ShapeDtypeStruct((e, hdim), F32),
            jax.ShapeDtypeStruct((e, 128), F32),
        ],
    )(g2, g2, ea, wr, we, w2, b2, watt, batt, wc1, bc1, wc2)


def _node_tc(h, agg, tsum, coord, wn1h, wn1a, bn1, wn2, bn2, bn):
    n, hdim = h.shape
    p = agg.shape[0] // n
    nb = n // bn

    def body(*refs):
        h_ref = refs[0]
        aggs = refs[1:1 + p]
        t0, t1 = refs[1 + p], refs[2 + p]
        coord_ref = refs[3 + p]
        wn1h_ref, wn1a_ref, bn1_ref, wn2_ref, bn2_ref = refs[4 + p:9 + p]
        h_out, c_out = refs[9 + p], refs[10 + p]

        a = aggs[0][...]
        if p == 2:
            a = a + aggs[1][...]
        ts = t0[...] + t1[...]
        lane = lax.broadcasted_iota(jnp.int32, (1, 128), 1)
        mask3 = (lane < 3).astype(F32)
        e3 = (lane == 3).astype(F32)
        cnt = jnp.sum(ts * e3, axis=1, keepdims=True)
        inv = 1.0 / jnp.maximum(cnt, 1.0)
        c_out[...] = coord_ref[...] + ts * mask3 * inv
        tmp = (_bdot(h_ref[...], wn1h_ref[...])
               + _bdot(a, wn1a_ref[...])
               + bn1_ref[...])
        tmp = _silu(tmp)
        h_out[...] = (h_ref[...]
                      + _bdot(tmp, wn2_ref[...])
                      + bn2_ref[...])

    cw = lambda shp: pl.BlockSpec(shp, lambda i: tuple(0 for _ in shp))
    in_specs = [pl.BlockSpec((bn, hdim), lambda i: (i, 0))]
    for q in range(p):
        in_specs.append(pl.BlockSpec((bn, hdim), lambda i, q=q: (q * nb + i, 0)))
    in_specs += [
        pl.BlockSpec((bn, 128), lambda i: (i, 0)),
        pl.BlockSpec((bn, 128), lambda i: (nb + i, 0)),
        pl.BlockSpec((bn, 128), lambda i: (i, 0)),
        cw((hdim, hdim)), cw((hdim, hdim)), cw((1, hdim)),
        cw((hdim, hdim)), cw((1, hdim)),
    ]
    return pl.pallas_call(
        body,
        grid=(nb,),
        in_specs=in_specs,
        out_specs=[
            pl.BlockSpec((bn, hdim), lambda i: (i, 0)),
            pl.BlockSpec((bn, 128), lambda i: (i, 0)),
        ],
        out_shape=[
            jax.ShapeDtypeStruct((n, hdim), F32),
            jax.ShapeDtypeStruct((n, 128), F32),
        ],
    )(h, *([agg] * p), tsum, tsum, coord, wn1h, wn1a, bn1.reshape(1, -1),
      wn2, bn2.reshape(1, -1))


def _final_tc(h, w, b, bn):
    n, k = h.shape
    m = w.shape[1]
    nb = n // bn

    def body(h_ref, w_ref, b_ref, o_ref, acc_ref):
        i = pl.program_id(0)

        @pl.when(i == 0)
        def _():
            acc_ref[...] = jnp.zeros_like(acc_ref)

        acc_ref[...] += jnp.sum(h_ref[...], axis=0, keepdims=True)

        @pl.when(i == nb - 1)
        def _():
            o_ref[...] = (
                _bdot(acc_ref[...] / n, w_ref[...])
                + b_ref[...]
            )

    return pl.pallas_call(
        body,
        grid=(nb,),
        in_specs=[
            pl.BlockSpec((bn, k), lambda i: (i, 0)),
            pl.BlockSpec((k, m), lambda i: (0, 0)),
            pl.BlockSpec((1, m), lambda i: (0, 0)),
        ],
        out_specs=pl.BlockSpec((1, m), lambda i: (0, 0)),
        out_shape=jax.ShapeDtypeStruct((1, m), F32),
        scratch_shapes=[pltpu.VMEM((1, k), F32)],
    )(h, w, b.reshape(1, -1))


def _sc_gather(table, idx, ch):
    v, d = table.shape
    dt = table.dtype
    b = idx.shape[0]
    assert b % NW == 0
    bpw = b // NW
    assert bpw % ch == 0 and ch % 8 == 0
    nit = bpw // ch
    assert nit >= 4
    mesh = plsc.VectorSubcoreMesh(core_axis_name="c", subcore_axis_name="s")

    @functools.partial(
        pl.kernel,
        mesh=mesh,
        out_type=jax.ShapeDtypeStruct((b, d), dt),
        scratch_types=[
            pltpu.VMEM((bpw,), jnp.int32),
            pltpu.VMEM((ch, d), dt),
            pltpu.VMEM((ch, d), dt),
            pltpu.SemaphoreType.DMA,
            pltpu.SemaphoreType.DMA,
            pltpu.SemaphoreType.DMA,
            pltpu.SemaphoreType.DMA,
        ],
    )
    def k(table_hbm, idx_hbm, out_hbm, idx_v, r0, r1, g0, g1, w0, w1):
        wid = lax.axis_index("s") * NC + lax.axis_index("c")
        base0 = pl.multiple_of(wid * bpw, 8)
        rows = (r0, r1)
        gs = (g0, g1)
        ws = (w0, w1)
        pltpu.sync_copy(idx_hbm.at[pl.ds(base0, bpw)], idx_v)
        for bb in range(2):
            pltpu.async_copy(
                table_hbm.at[idx_v.at[pl.ds(bb * ch, ch)]], rows[bb], gs[bb])

        def step(j, carry):
            for bb in range(2):
                i = 2 * j + bb
                pltpu.make_async_copy(
                    table_hbm.at[idx_v.at[pl.ds(0, ch)]], rows[bb],
                    gs[bb]).wait()
                off = pl.multiple_of(base0 + i * ch, 8)
                pltpu.async_copy(rows[bb], out_hbm.at[pl.ds(off, ch)], ws[bb])

                @pl.when(i + 2 < nit)
                def _():
                    pltpu.make_async_copy(
                        rows[bb], out_hbm.at[pl.ds(base0, ch)], ws[bb]).wait()
                    pltpu.async_copy(
                        table_hbm.at[idx_v.at[pl.ds((i + 2) * ch, ch)]],
                        rows[bb], gs[bb])

            return carry

        lax.fori_loop(0, nit // 2, step, 0)
        if nit % 2 == 1:
            pltpu.make_async_copy(
                table_hbm.at[idx_v.at[pl.ds(0, ch)]], rows[0], gs[0]).wait()
            off = pl.multiple_of(base0 + (nit - 1) * ch, 8)
            pltpu.async_copy(rows[0], out_hbm.at[pl.ds(off, ch)], ws[0])
        for bb in range(2):
            pltpu.make_async_copy(
                rows[bb], out_hbm.at[pl.ds(base0, ch)], ws[bb]).wait()

    return k(table, idx)


def _sc_scatter_cols(mlo, mhi, idx, zeros, n, c, bet):
    e2, hdim = mlo.shape
    e = 2 * e2
    nck = hdim // c
    ncps = nck // NC
    ept = e // NS
    assert ept % bet == 0 and bet % 8 == 0
    nit = ept // bet
    rpt = (-(-n // NS) + 7) // 8 * 8
    tail = n - (NS - 1) * rpt
    assert tail > 0 and tail % 8 == 0
    mesh = plsc.VectorSubcoreMesh(core_axis_name="c", subcore_axis_name="s")

    @functools.partial(
        pl.kernel,
        mesh=mesh,
        out_type=jax.ShapeDtypeStruct((n, hdim), F32),
        scratch_types=[
            pltpu.VMEM((bet, c), F32),
            pltpu.VMEM((bet, c), F32),
            pltpu.VMEM((bet,), jnp.int32),
            pltpu.VMEM((bet,), jnp.int32),
            pltpu.VMEM_SHARED((n, c), F32),
            pltpu.SemaphoreType.DMA,
            pltpu.SemaphoreType.DMA,
            pltpu.SemaphoreType.DMA,
            pltpu.SemaphoreType.DMA,
        ],
    )
    def k(mlo_hbm, mhi_hbm, idx_hbm, z_hbm, out_hbm, m0, m1, i0, i1, acc,
          l0, l1, s0, s1):
        core = lax.axis_index("c")
        s = lax.axis_index("s")
        ns2 = NS // 2
        r0 = pl.multiple_of(s * rpt, 8)
        mb = (m0, m1)
        ib = (i0, i1)
        ls = (l0, l1)
        ss = (s0, s1)
        ebase = pl.multiple_of(s * ept, 8)
        lbase = pl.multiple_of(jnp.where(s < ns2, s, s - ns2) * ept, 8)

        def fire_load(i, bb, c0):
            e0 = pl.multiple_of(ebase + i * bet, 8)
            e0l = pl.multiple_of(lbase + i * bet, 8)
            pltpu.async_copy(idx_hbm.at[pl.ds(e0, bet)], ib[bb], ls[bb])

            @pl.when(s < ns2)
            def _():
                pltpu.async_copy(mlo_hbm.at[pl.ds(e0l, bet), pl.ds(c0, c)],
                                 mb[bb], ls[bb])

            @pl.when(s >= ns2)
            def _():
                pltpu.async_copy(mhi_hbm.at[pl.ds(e0l, bet), pl.ds(c0, c)],
                                 mb[bb], ls[bb])

        for kk in range(ncps):
            chunk = core + NC * kk
            c0 = chunk * c

            @pl.when(s < NS - 1)
            def _():
                pltpu.sync_copy(z_hbm.at[pl.ds(0, rpt), pl.ds(0, c)],
                                acc.at[pl.ds(r0, rpt)])

            @pl.when(s == NS - 1)
            def _():
                pltpu.sync_copy(z_hbm.at[pl.ds(0, tail), pl.ds(0, c)],
                                acc.at[pl.ds(r0, tail)])

            plsc.subcore_barrier()
            for bb in range(2):
                fire_load(bb, bb, c0)

            def step(j, carry):
                for bb in range(2):
                    i = 2 * j + bb
                    pltpu.make_async_copy(
                        idx_hbm.at[pl.ds(0, bet)], ib[bb], ls[bb]).wait()
                    pltpu.make_async_copy(
                        mlo_hbm.at[pl.ds(0, bet), pl.ds(0, c)], mb[bb],
                        ls[bb]).wait()
                    pltpu.async_copy(mb[bb], acc.at[ib[bb]], ss[bb],
                                     add=True)

                    @pl.when(i + 2 < nit)
                    def _():
                        pltpu.make_async_copy(
                            mb[bb], acc.at[ib[bb]], ss[bb]).wait()
                        fire_load(i + 2, bb, c0)

                return carry

            lax.fori_loop(0, nit // 2, step, 0)
            if nit % 2 == 0:
                for bb in range(2):
                    pltpu.make_async_copy(mb[bb], acc.at[ib[bb]],
                                          ss[bb]).wait()
            else:
                pltpu.make_async_copy(mb[1], acc.at[ib[1]], ss[1]).wait()
                pltpu.make_async_copy(
                    idx_hbm.at[pl.ds(0, bet)], ib[0], ls[0]).wait()
                pltpu.make_async_copy(
                    mlo_hbm.at[pl.ds(0, bet), pl.ds(0, c)], mb[0],
                    ls[0]).wait()
                pltpu.async_copy(mb[0], acc.at[ib[0]], ss[0], add=True)
                pltpu.make_async_copy(mb[0], acc.at[ib[0]], ss[0]).wait()
            plsc.subcore_barrier()

            @pl.when(s < NS - 1)
            def _():
                pltpu.sync_copy(acc.at[pl.ds(r0, rpt)],
                                out_hbm.at[pl.ds(r0, rpt), pl.ds(c0, c)])

            @pl.when(s == NS - 1)
            def _():
                pltpu.sync_copy(acc.at[pl.ds(r0, tail)],
                                out_hbm.at[pl.ds(r0, tail), pl.ds(c0, c)])

            plsc.subcore_barrier()

    return k(mlo, mhi, idx, zeros)


def _sc_scatter_halves(mlo, mhi, idx, zeros, n, bet):
    e2, c = mlo.shape
    e = 2 * e2
    eps = e // NC
    ept = eps // NS
    assert ept % bet == 0 and bet % 8 == 0
    nit = ept // bet
    rpt = (-(-n // NS) + 7) // 8 * 8
    tail = n - (NS - 1) * rpt
    assert tail > 0 and tail % 8 == 0
    mesh = plsc.VectorSubcoreMesh(core_axis_name="c", subcore_axis_name="s")

    assert nit >= 4
    npipe = nit // 2 * 2

    @functools.partial(
        pl.kernel,
        mesh=mesh,
        out_type=jax.ShapeDtypeStruct((2 * n, c), F32),
        scratch_types=[
            pltpu.VMEM((bet, c), F32),
            pltpu.VMEM((bet, c), F32),
            pltpu.VMEM((bet,), jnp.int32),
            pltpu.VMEM((bet,), jnp.int32),
            pltpu.VMEM_SHARED((n, c), F32),
            pltpu.SemaphoreType.DMA,
            pltpu.SemaphoreType.DMA,
            pltpu.SemaphoreType.DMA,
            pltpu.SemaphoreType.DMA,
        ],
    )
    def k(mlo_hbm, mhi_hbm, idx_hbm, z_hbm, out_hbm, m0, m1, i0, i1, acc,
          l0, l1, s0, s1):
        core = lax.axis_index("c")
        s = lax.axis_index("s")
        r0 = pl.multiple_of(s * rpt, 8)
        o0 = pl.multiple_of(core * n + s * rpt, 8)
        mb = (m0, m1)
        ib = (i0, i1)
        ls = (l0, l1)
        ss = (s0, s1)
        ebase = pl.multiple_of(core * eps + s * ept, 8)
        lbase = pl.multiple_of(s * ept, 8)

        def fire_load(i, bb):
            e0 = pl.multiple_of(ebase + i * bet, 8)
            e0l = pl.multiple_of(lbase + i * bet, 8)
            pltpu.async_copy(idx_hbm.at[pl.ds(e0, bet)], ib[bb], ls[bb])

            @pl.when(core == 0)
            def _():
                pltpu.async_copy(mlo_hbm.at[pl.ds(e0l, bet), pl.ds(0, c)],
                                 mb[bb], ls[bb])

            @pl.when(core == 1)
            def _():
                pltpu.async_copy(mhi_hbm.at[pl.ds(e0l, bet), pl.ds(0, c)],
                                 mb[bb], ls[bb])

        def wait_load(bb):
            pltpu.make_async_copy(
                idx_hbm.at[pl.ds(0, bet)], ib[bb], ls[bb]).wait()
            pltpu.make_async_copy(
                mlo_hbm.at[pl.ds(0, bet), pl.ds(0, c)], mb[bb],
                ls[bb]).wait()

        @pl.when(s < NS - 1)
        def _():
            pltpu.sync_copy(z_hbm.at[pl.ds(0, rpt), pl.ds(0, c)],
                            acc.at[pl.ds(r0, rpt)])

        @pl.when(s == NS - 1)
        def _():
            pltpu.sync_copy(z_hbm.at[pl.ds(0, tail), pl.ds(0, c)],
                            acc.at[pl.ds(r0, tail)])

        plsc.subcore_barrier()
        for bb in range(2):
            fire_load(bb, bb)

        def step(j, carry):
            for bb in range(2):
                i = 2 * j + bb
                wait_load(bb)
                pltpu.async_copy(mb[bb], acc.at[ib[bb]], ss[bb], add=True)

                @pl.when(i + 2 < nit)
                def _():
                    pltpu.make_async_copy(mb[bb], acc.at[ib[bb]],
                                          ss[bb]).wait()
                    fire_load(i + 2, bb)

            return carry

        lax.fori_loop(0, nit // 2, step, 0)
        if npipe == nit:
            for bb in range(2):
                pltpu.make_async_copy(mb[bb], acc.at[ib[bb]], ss[bb]).wait()
        else:
            pltpu.make_async_copy(mb[1], acc.at[ib[1]], ss[1]).wait()
            wait_load(0)
            pltpu.async_copy(mb[0], acc.at[ib[0]], ss[0], add=True)
            pltpu.make_async_copy(mb[0], acc.at[ib[0]], ss[0]).wait()
        plsc.subcore_barrier()

        @pl.when(s < NS - 1)
        def _():
            pltpu.sync_copy(acc.at[pl.ds(r0, rpt)],
                            out_hbm.at[pl.ds(o0, rpt), pl.ds(0, c)])

        @pl.when(s == NS - 1)
        def _():
            pltpu.sync_copy(acc.at[pl.ds(r0, tail)],
                            out_hbm.at[pl.ds(o0, tail), pl.ds(0, c)])

    return k(mlo, mhi, idx, zeros)


def _egcl(h, coord, lp, idx_h, row, ea_h, zeros, n, e, hdim,
          bn, be, ch_pq, bet_m, cols_mode):
    w1 = lp["edge1"]["w"]
    wa, wb = w1[:hdim], w1[hdim:2 * hdim]
    wr = w1[2 * hdim:2 * hdim + 1]
    we = jnp.pad(w1[2 * hdim + 1:], ((0, 5), (0, 0)))
    b1 = lp["edge1"]["b"]

    pq = _pq_tc(h, jnp.stack([wa, wb]),
                jnp.stack([b1.reshape(1, -1), jnp.zeros((1, hdim), F32)]),
                coord, bn)
    g0 = _sc_gather(pq, idx_h[0], ch_pq)
    g1 = _sc_gather(pq, idx_h[1], ch_pq)

    ew = (lp["edge2"]["w"], lp["edge2"]["b"].reshape(1, -1),
          lp["att"]["w"].reshape(1, -1), lp["att"]["b"].reshape(1, 1),
          lp["coord1"]["w"], lp["coord1"]["b"].reshape(1, -1),
          lp["coord2_w"].reshape(1, -1))
    m0, t0 = _edge_tc(g0, ea_h[0], wr, we, *ew, e // 2, hdim, be)
    m1, t1 = _edge_tc(g1, ea_h[1], wr, we, *ew, e // 2, hdim, be)

    if cols_mode:
        agg = _sc_scatter_cols(m0, m1, row, zeros, n, 128, bet_m)
    else:
        agg = _sc_scatter_halves(m0, m1, row, zeros, n, bet_m)
    tsum = _sc_scatter_halves(t0, t1, row, zeros, n, 40)

    wn1 = lp["node1"]["w"]
    h, coord = _node_tc(h, agg, tsum, coord, wn1[:hdim], wn1[hdim:],
                        lp["node1"]["b"], lp["node2"]["w"], lp["node2"]["b"],
                        bn)
    return h, coord


def kernel(node_features, edge_index, node_pos, edge_attr, params):
    n = node_features.shape[0]
    e = edge_index.shape[1]
    row = edge_index[0]
    col = edge_index[1]
    e2 = e // 2
    idx_h = (jnp.concatenate([row[:e2], col[:e2] + n]),
             jnp.concatenate([row[e2:], col[e2:] + n]))
    ea16 = jnp.pad(edge_attr, ((0, 0), (0, 16 - edge_attr.shape[1])))
    ea_h = (ea16[:e2], ea16[e2:])
    zeros = jnp.zeros((n, 128), F32)
    coord = jnp.pad(node_pos, ((0, 0), (0, 125)))
    bn = 1000

    p1 = params["egnn1"]
    h = _linear_tc(node_features, p1["emb_in"]["w"], p1["emb_in"]["b"], bn)
    for lp in p1["layers"]:
        h, coord = _egcl(h, coord, lp, idx_h, row, ea_h, zeros,
                         n, e, 512, bn, 640, 40, 80, cols_mode=True)
    h = _linear_tc(h, p1["emb_out"]["w"], p1["emb_out"]["b"], bn)

    p2 = params["egnn2"]
    h = _linear_tc(h, p2["emb_in"]["w"], p2["emb_in"]["b"], bn)
    for lp in p2["layers"]:
        h, coord = _egcl(h, coord, lp, idx_h, row, ea_h, zeros,
                         n, e, 128, bn, 2000, 200, 40, cols_mode=False)
    out = _final_tc(h, p2["emb_out"]["w"], p2["emb_out"]["b"], bn)
    return out
